# baseline (speedup 1.0000x reference)
import jax
import jax.numpy as jnp
from jax.experimental import pallas as pl
from jax.experimental.pallas import tpu as pltpu

_N = 10000
_DIN = 128
_DH = 64
_BLOCK = 1000


def _cell_kernel(x_ref, h_ref, c_ref, W_ref, Th_ref, b_ref, wci_ref, wcf_ref,
                 wco_ref, wfc_ref, bfc_ref, out_ref, H_ref, C_ref):
    x = x_ref[...]
    h = h_ref[...]
    c = c_ref[...]
    z = (jnp.dot(x, W_ref[...], preferred_element_type=jnp.float32)
         + jnp.dot(h, Th_ref[...], preferred_element_type=jnp.float32)
         + b_ref[...])
    zi = z[:, 0 * _DH:1 * _DH]
    zf = z[:, 1 * _DH:2 * _DH]
    zc = z[:, 2 * _DH:3 * _DH]
    zo = z[:, 3 * _DH:4 * _DH]
    I = jax.nn.sigmoid(zi + wci_ref[...] * c)
    F = jax.nn.sigmoid(zf + wcf_ref[...] * c)
    T = jnp.tanh(zc)
    C = F * c + I * T
    O = jax.nn.sigmoid(zo + wco_ref[...] * C)
    H = O * jnp.tanh(C)
    C_ref[...] = C
    H_ref[...] = H
    out_ref[...] = (jnp.sum(jax.nn.relu(H) * wfc_ref[...], axis=1,
                            keepdims=True) + bfc_ref[...])


def kernel(x, edge_index, edge_weight, h, c, W_i, W_f, W_c, W_o, Th_i, bh_i,
           Th_f, bh_f, Th_c, bh_c, Th_o, bh_o, w_ci, w_cf, w_co, b_i, b_f,
           b_c, b_o, W_fc, b_fc):
    del edge_index, edge_weight
    W = jnp.concatenate([W_i, W_f, W_c, W_o], axis=1)
    Th = jnp.concatenate([Th_i, Th_f, Th_c, Th_o], axis=1)
    b = jnp.concatenate([bh_i[None, :] + b_i, bh_f[None, :] + b_f,
                         bh_c[None, :] + b_c, bh_o[None, :] + b_o],
                        axis=1)
    wfc = W_fc.reshape(1, _DH)
    bfc = b_fc.reshape(1, 1)

    grid = (_N // _BLOCK,)
    row = lambda i: (i, 0)
    rep = lambda i: (0, 0)
    out, H, C = pl.pallas_call(
        _cell_kernel,
        grid=grid,
        in_specs=[
            pl.BlockSpec((_BLOCK, _DIN), row),
            pl.BlockSpec((_BLOCK, _DH), row),
            pl.BlockSpec((_BLOCK, _DH), row),
            pl.BlockSpec((_DIN, 4 * _DH), rep),
            pl.BlockSpec((_DH, 4 * _DH), rep),
            pl.BlockSpec((1, 4 * _DH), rep),
            pl.BlockSpec((1, _DH), rep),
            pl.BlockSpec((1, _DH), rep),
            pl.BlockSpec((1, _DH), rep),
            pl.BlockSpec((1, _DH), rep),
            pl.BlockSpec((1, 1), rep),
        ],
        out_specs=[
            pl.BlockSpec((_BLOCK, 1), row),
            pl.BlockSpec((_BLOCK, _DH), row),
            pl.BlockSpec((_BLOCK, _DH), row),
        ],
        out_shape=[
            jax.ShapeDtypeStruct((_N, 1), jnp.float32),
            jax.ShapeDtypeStruct((_N, _DH), jnp.float32),
            jax.ShapeDtypeStruct((_N, _DH), jnp.float32),
        ],
        compiler_params=pltpu.CompilerParams(
            dimension_semantics=("arbitrary",),
        ),
    )(x, h, c, W, Th, b, w_ci, w_cf, w_co, wfc, bfc)
    return (out, H, C)
